# baseline (speedup 1.0000x reference)
import functools

import numpy as np
import jax
import jax.numpy as jnp
from jax import lax
from jax.experimental import pallas as pl
from jax.experimental.pallas import tpu as pltpu
from jax.experimental.pallas import tpu_sc as plsc

T = 2048
D = 1024
FF = 1024
E = 64
K = 8
ROUTE_SCALE = 2.826

M = 128
PAD_T = 24576
NT = PAD_T // M
NTE = 512
I_NT = 192
I_NR = 193
I_RUNE = 200
I_ROT = 320
NBUF = 4

SC_NC = 2
SC_NS = 16
SC_NW = SC_NC * SC_NS
PAIRS = T * K
PPW = PAIRS // SC_NW
CH = 32
NCH = PPW // CH

_TOK = np.repeat(np.arange(T, dtype=np.int32), K)


def _ct(a, b):
    return jax.lax.dot_general(
        a, b, (((1,), (1,)), ((), ())), preferred_element_type=jnp.float32
    )


def _mm(a, b):
    return jax.lax.dot_general(
        a, b, (((1,), (0,)), ((), ())), preferred_element_type=jnp.float32
    )


def _router_body(x_ref, wr_ref, bias_ref, w_ref, pos_ref, te_ref):
    x = x_ref[...]
    gates = _ct(x, wr_ref[...])
    scores = jax.nn.sigmoid(gates)
    selection = scores + bias_ref[...]
    iota = lax.broadcasted_iota(jnp.int32, (T, E), 1)
    remaining = selection
    masks = []
    for _ in range(K):
        m = jnp.max(remaining, axis=1, keepdims=True)
        first = jnp.min(jnp.where(remaining == m, iota, E), axis=1, keepdims=True)
        mk = (iota == first).astype(jnp.float32)
        masks.append(mk)
        remaining = jnp.where(mk > 0, -jnp.inf, remaining)
    picked = masks[0]
    for mk in masks[1:]:
        picked = picked + mk
    wsum = jnp.sum(scores * picked, axis=1, keepdims=True)
    wall = scores * (ROUTE_SCALE / wsum)
    w_ref[...] = jnp.concatenate(
        [jnp.sum(mk * wall, axis=1, keepdims=True) for mk in masks], axis=1
    )

    CB = 256
    tri = (
        lax.broadcasted_iota(jnp.int32, (CB, CB), 0)
        > lax.broadcasted_iota(jnp.int32, (CB, CB), 1)
    ).astype(jnp.float32)
    carry = jnp.zeros((1, E), jnp.float32)
    parts = []
    for c in range(T // CB):
        blk = picked[c * CB : (c + 1) * CB, :]
        parts.append(_mm(tri, blk) + carry)
        carry = carry + jnp.sum(blk, axis=0, keepdims=True)
    csum_excl = jnp.concatenate(parts, axis=0)

    pc_row = jnp.ceil(carry / M) * M
    sltE = (
        lax.broadcasted_iota(jnp.int32, (E, E), 0)
        < lax.broadcasted_iota(jnp.int32, (E, E), 1)
    ).astype(jnp.float32)
    po_lane = _mm(pc_row, sltE)
    field = po_lane + csum_excl
    pos_f = jnp.concatenate(
        [jnp.sum(mk * field, axis=1, keepdims=True) for mk in masks], axis=1
    )
    pos_ref[...] = pos_f.astype(jnp.int32)

    ones_t = jnp.ones((T, 1), jnp.float32)
    counts_sub = jax.lax.dot_general(
        picked, ones_t, (((0,), (0,)), ((), ())), preferred_element_type=jnp.float32
    )
    pc_sub = jnp.ceil(counts_sub / M) * M
    slo = (
        lax.broadcasted_iota(jnp.int32, (E, E), 1)
        < lax.broadcasted_iota(jnp.int32, (E, E), 0)
    ).astype(jnp.float32)
    po_sub = _mm(slo, pc_sub)
    lane = lax.broadcasted_iota(jnp.int32, (1, NTE), 1)
    ones_e = jnp.ones((1, E), jnp.float32)
    jm = (lane * M).astype(jnp.float32)
    amat = (po_sub <= jm).astype(jnp.float32)
    te = _mm(ones_e, amat) - 1.0
    nonempty = (counts_sub > 0).astype(jnp.float32)
    sle = (
        lax.broadcasted_iota(jnp.int32, (E, E), 1)
        <= lax.broadcasted_iota(jnp.int32, (E, E), 0)
    ).astype(jnp.float32)
    rank_sub = _mm(sle, nonempty)
    nruns_val = _mm(ones_e, nonempty)
    a_rune = nonempty * (
        (rank_sub - 1.0) == (lane - I_RUNE).astype(jnp.float32)
    ).astype(jnp.float32)
    e_row = lax.broadcasted_iota(jnp.int32, (1, E), 1).astype(jnp.float32)
    rune = _mm(e_row, a_rune)
    jm2 = ((lane - I_ROT) * M).astype(jnp.float32)
    a_rot = nonempty * (po_sub <= jm2).astype(jnp.float32)
    rot = _mm(ones_e, a_rot) - 1.0
    mask63 = (lax.broadcasted_iota(jnp.int32, (E, 1), 0) == E - 1).astype(jnp.float32)
    nt_val = _mm(ones_e, (po_sub + pc_sub) * mask63) / M
    te = jnp.where(lane == I_NT, nt_val, te)
    te = jnp.where(lane == I_NR, nruns_val, te)
    te = jnp.where((lane >= I_RUNE) & (lane < I_RUNE + E), rune, te)
    te = jnp.where(lane >= I_ROT, rot, te)
    te_ref[...] = te.astype(jnp.int32)


def _router_meta(x, Wr, expert_bias):
    return pl.pallas_call(
        _router_body,
        out_shape=(
            jax.ShapeDtypeStruct((T, K), jnp.float32),
            jax.ShapeDtypeStruct((T, K), jnp.int32),
            jax.ShapeDtypeStruct((1, NTE), jnp.int32),
        ),
    )(x, Wr, expert_bias.reshape(1, E))


TPW = T // SC_NW


@functools.cache
def _make_scatter_x():
    @functools.partial(
        pl.kernel,
        mesh=plsc.VectorSubcoreMesh(
            core_axis_name="c", subcore_axis_name="s", num_cores=SC_NC,
            num_subcores=SC_NS,
        ),
        out_type=jax.ShapeDtypeStruct((PAD_T, D), jnp.float32),
        scratch_types=[
            pltpu.VMEM((K, TPW), jnp.int32),
            pltpu.VMEM((TPW, D), jnp.float32),
            pltpu.SemaphoreType.DMA,
            pltpu.SemaphoreType.DMA,
            pltpu.SemaphoreType.DMA,
        ],
    )
    def _scatter_x(x_hbm, post_hbm, xs_hbm, idx_v, rows_v, gsem, s0, s1):
        wid = lax.axis_index("s") * SC_NC + lax.axis_index("c")
        pltpu.sync_copy(post_hbm.at[wid], idx_v)
        pltpu.async_copy(x_hbm.at[pl.ds(wid * TPW, TPW)], rows_v, gsem).wait()
        ss = [s0, s1]
        hs = [
            pltpu.async_copy(rows_v, xs_hbm.at[idx_v.at[k]], ss[k & 1])
            for k in range(K)
        ]
        for h in hs:
            h.wait()

    return _scatter_x


def _ffn_body(te_ref, x_ref, wg_hbm, wu_hbm, wd_hbm, y_ref, wgb, wub, wdb, sems):
    i = pl.program_id(0)
    nruns = te_ref[I_NR]
    r = te_ref[I_ROT + i]

    def dmas(rp, slot):
        e = te_ref[I_RUNE + rp]
        return (
            pltpu.make_async_copy(wg_hbm.at[e], wgb.at[slot], sems.at[slot, 0]),
            pltpu.make_async_copy(wu_hbm.at[e], wub.at[slot], sems.at[slot, 1]),
            pltpu.make_async_copy(wd_hbm.at[e], wdb.at[slot], sems.at[slot, 2]),
        )

    def issue(rp):
        for d in dmas(rp, lax.rem(rp, NBUF)):
            d.start()

    def drain(rp):
        for d in dmas(rp, lax.rem(rp, NBUF)):
            d.wait()

    @pl.when(i == 0)
    def _():
        issue(0)

    @pl.when((i == 0) & (nruns > 1))
    def _():
        issue(1)

    @pl.when((i == 0) & (nruns > 2))
    def _():
        issue(2)

    @pl.when((i == 0) & (nruns > 3))
    def _():
        issue(3)

    @pl.when(i == 0)
    def _():
        drain(0)

    first = (i > 0) & (r != te_ref[I_ROT + i - 1])

    @pl.when(first & (r + NBUF - 1 < nruns))
    def _():
        issue(r + NBUF - 1)

    @pl.when(first)
    def _():
        drain(r)

    @pl.when(i < te_ref[I_NT])
    def _():
        slot = lax.rem(r, NBUF)
        x = x_ref[...]
        h = jax.nn.silu(_ct(x, wgb[slot])) * _ct(x, wub[slot])
        y_ref[...] = _ct(h, wdb[slot])


def _grouped_ffn(te, xs, Wg, Wu, Wd):
    def clamp(i, te):
        return jnp.minimum(i, te[I_NT] - 1)

    grid_spec = pltpu.PrefetchScalarGridSpec(
        num_scalar_prefetch=1,
        grid=(NT,),
        in_specs=[
            pl.BlockSpec((M, D), lambda i, te: (clamp(i, te), 0)),
            pl.BlockSpec(memory_space=pl.ANY),
            pl.BlockSpec(memory_space=pl.ANY),
            pl.BlockSpec(memory_space=pl.ANY),
        ],
        out_specs=pl.BlockSpec((M, D), lambda i, te: (clamp(i, te), 0)),
        scratch_shapes=[
            pltpu.VMEM((NBUF, FF, D), jnp.float32),
            pltpu.VMEM((NBUF, FF, D), jnp.float32),
            pltpu.VMEM((NBUF, D, FF), jnp.float32),
            pltpu.SemaphoreType.DMA((NBUF, 3)),
        ],
    )
    return pl.pallas_call(
        _ffn_body,
        grid_spec=grid_spec,
        out_shape=jax.ShapeDtypeStruct((PAD_T, D), jnp.float32),
        compiler_params=pltpu.CompilerParams(
            dimension_semantics=("arbitrary",),
        ),
    )(te, xs, Wg, Wu, Wd)


@functools.cache
def _make_gather_y():
    @functools.partial(
        pl.kernel,
        mesh=plsc.VectorSubcoreMesh(
            core_axis_name="c", subcore_axis_name="s", num_cores=SC_NC,
            num_subcores=SC_NS,
        ),
        out_type=jax.ShapeDtypeStruct((K, T, D), jnp.float32),
        scratch_types=[
            pltpu.VMEM((2 * K, TPW // 2), jnp.int32),
            pltpu.VMEM((2, TPW // 2, D), jnp.float32),
            pltpu.SemaphoreType.DMA,
            pltpu.SemaphoreType.DMA,
            pltpu.SemaphoreType.DMA,
            pltpu.SemaphoreType.DMA,
        ],
    )
    def _gather_y(ys_hbm, post_hbm, yp_hbm, idx_v, rows_v, g0, g1, s0, s1):
        wid = lax.axis_index("s") * SC_NC + lax.axis_index("c")
        tok0 = wid * TPW
        half = TPW // 2
        pltpu.sync_copy(post_hbm.at[wid], idx_v)
        gs, ss = [g0, g1], [s0, s1]

        def gather(j, b):
            return pltpu.async_copy(ys_hbm.at[idx_v.at[j]], rows_v.at[b], gs[b])

        def put(j, b):
            k, h = j // 2, j % 2
            return pltpu.async_copy(
                rows_v.at[b], yp_hbm.at[k, pl.ds(tok0 + h * half, half)], ss[b]
            )

        n = 2 * K
        gh, sh = {}, {}
        gh[0] = gather(0, 0)
        for j in range(n):
            b = j & 1
            if j + 1 < n:
                if j - 1 >= 0:
                    sh[j - 1].wait()
                gh[j + 1] = gather(j + 1, 1 - b)
            gh[j].wait()
            sh[j] = put(j, b)
        sh[n - 2].wait()
        sh[n - 1].wait()

    return _gather_y


TB = 256


def _combine_body(yp_ref, w_ref, x_ref, gs_ref, us_ref, ds_ref, y_ref):
    x = x_ref[...]
    acc = _ct(jax.nn.silu(_ct(x, gs_ref[...])) * _ct(x, us_ref[...]), ds_ref[...])
    for k in range(K):
        acc = acc + w_ref[:, k : k + 1] * yp_ref[k]
    y_ref[...] = acc


def _combine(yp, w, x, Gs, Us, Ds):
    full = lambda shape: pl.BlockSpec(shape, lambda t: (0,) * len(shape))
    return pl.pallas_call(
        _combine_body,
        grid=(T // TB,),
        in_specs=[
            pl.BlockSpec((K, TB, D), lambda t: (0, t, 0)),
            pl.BlockSpec((TB, K), lambda t: (t, 0)),
            pl.BlockSpec((TB, D), lambda t: (t, 0)),
            full((FF, D)),
            full((FF, D)),
            full((D, FF)),
        ],
        out_specs=pl.BlockSpec((TB, D), lambda t: (t, 0)),
        out_shape=jax.ShapeDtypeStruct((T, D), jnp.float32),
        compiler_params=pltpu.CompilerParams(
            dimension_semantics=("arbitrary",),
        ),
    )(yp, w, x, Gs, Us, Ds)


_TOK2 = _TOK.reshape(PAIRS // CH, CH)


@jax.jit
def kernel(x, Wr, expert_bias, Wg, Wu, Wd, Gs, Us, Ds):
    w, pos, te = _router_meta(x, Wr, expert_bias)
    post = pos.reshape(SC_NW, TPW, K).transpose(0, 2, 1)
    tef = te.reshape(NTE)
    xs = _make_scatter_x()(x, post)
    ys = _grouped_ffn(tef, xs, Wg, Wu, Wd)
    post16 = post.reshape(SC_NW, 2 * K, TPW // 2)
    yp = _make_gather_y()(ys, post16)
    return _combine(yp, w, x, Gs, Us, Ds)


# speedup vs baseline: 1.0163x; 1.0163x over previous
import functools

import numpy as np
import jax
import jax.numpy as jnp
from jax import lax
from jax.experimental import pallas as pl
from jax.experimental.pallas import tpu as pltpu
from jax.experimental.pallas import tpu_sc as plsc

T = 2048
D = 1024
FF = 1024
E = 64
K = 8
ROUTE_SCALE = 2.826

M = 128
PAD_T = 24576
NT = PAD_T // M
NTE = 512
I_NT = 192
I_NR = 193
I_RUNE = 200
I_ROT = 320
NBUF = 3

SC_NC = 2
SC_NS = 16
SC_NW = SC_NC * SC_NS
PAIRS = T * K
PPW = PAIRS // SC_NW
CH = 32
NCH = PPW // CH

_TOK = np.repeat(np.arange(T, dtype=np.int32), K)


def _ct(a, b):
    return jax.lax.dot_general(
        a, b, (((1,), (1,)), ((), ())), preferred_element_type=jnp.float32
    )


def _mm(a, b):
    return jax.lax.dot_general(
        a, b, (((1,), (0,)), ((), ())), preferred_element_type=jnp.float32
    )


def _router_body(x_ref, wr_ref, bias_ref, w_ref, pos_ref, te_ref):
    x = x_ref[...]
    gates = _ct(x, wr_ref[...])
    scores = jax.nn.sigmoid(gates)
    selection = scores + bias_ref[...]
    iota = lax.broadcasted_iota(jnp.int32, (T, E), 1)
    remaining = selection
    masks = []
    for _ in range(K):
        m = jnp.max(remaining, axis=1, keepdims=True)
        first = jnp.min(jnp.where(remaining == m, iota, E), axis=1, keepdims=True)
        mk = (iota == first).astype(jnp.float32)
        masks.append(mk)
        remaining = jnp.where(mk > 0, -jnp.inf, remaining)
    picked = masks[0]
    for mk in masks[1:]:
        picked = picked + mk
    wsum = jnp.sum(scores * picked, axis=1, keepdims=True)
    wall = scores * (ROUTE_SCALE / wsum)
    w_ref[...] = jnp.concatenate(
        [jnp.sum(mk * wall, axis=1, keepdims=True) for mk in masks], axis=1
    )

    CB = 256
    tri = (
        lax.broadcasted_iota(jnp.int32, (CB, CB), 0)
        > lax.broadcasted_iota(jnp.int32, (CB, CB), 1)
    ).astype(jnp.float32)
    carry = jnp.zeros((1, E), jnp.float32)
    parts = []
    for c in range(T // CB):
        blk = picked[c * CB : (c + 1) * CB, :]
        parts.append(_mm(tri, blk) + carry)
        carry = carry + jnp.sum(blk, axis=0, keepdims=True)
    csum_excl = jnp.concatenate(parts, axis=0)

    pc_row = jnp.ceil(carry / M) * M
    sltE = (
        lax.broadcasted_iota(jnp.int32, (E, E), 0)
        < lax.broadcasted_iota(jnp.int32, (E, E), 1)
    ).astype(jnp.float32)
    po_lane = _mm(pc_row, sltE)
    field = po_lane + csum_excl
    pos_f = jnp.concatenate(
        [jnp.sum(mk * field, axis=1, keepdims=True) for mk in masks], axis=1
    )
    pos_ref[...] = pos_f.astype(jnp.int32)

    ones_t = jnp.ones((T, 1), jnp.float32)
    counts_sub = jax.lax.dot_general(
        picked, ones_t, (((0,), (0,)), ((), ())), preferred_element_type=jnp.float32
    )
    pc_sub = jnp.ceil(counts_sub / M) * M
    slo = (
        lax.broadcasted_iota(jnp.int32, (E, E), 1)
        < lax.broadcasted_iota(jnp.int32, (E, E), 0)
    ).astype(jnp.float32)
    po_sub = _mm(slo, pc_sub)
    lane = lax.broadcasted_iota(jnp.int32, (1, NTE), 1)
    ones_e = jnp.ones((1, E), jnp.float32)
    jm = (lane * M).astype(jnp.float32)
    amat = (po_sub <= jm).astype(jnp.float32)
    te = _mm(ones_e, amat) - 1.0
    nonempty = (counts_sub > 0).astype(jnp.float32)
    sle = (
        lax.broadcasted_iota(jnp.int32, (E, E), 1)
        <= lax.broadcasted_iota(jnp.int32, (E, E), 0)
    ).astype(jnp.float32)
    rank_sub = _mm(sle, nonempty)
    nruns_val = _mm(ones_e, nonempty)
    a_rune = nonempty * (
        (rank_sub - 1.0) == (lane - I_RUNE).astype(jnp.float32)
    ).astype(jnp.float32)
    e_row = lax.broadcasted_iota(jnp.int32, (1, E), 1).astype(jnp.float32)
    rune = _mm(e_row, a_rune)
    jm2 = ((lane - I_ROT) * M).astype(jnp.float32)
    a_rot = nonempty * (po_sub <= jm2).astype(jnp.float32)
    rot = _mm(ones_e, a_rot) - 1.0
    mask63 = (lax.broadcasted_iota(jnp.int32, (E, 1), 0) == E - 1).astype(jnp.float32)
    nt_val = _mm(ones_e, (po_sub + pc_sub) * mask63) / M
    te = jnp.where(lane == I_NT, nt_val, te)
    te = jnp.where(lane == I_NR, nruns_val, te)
    te = jnp.where((lane >= I_RUNE) & (lane < I_RUNE + E), rune, te)
    te = jnp.where(lane >= I_ROT, rot, te)
    te_ref[...] = te.astype(jnp.int32)


def _router_meta(x, Wr, expert_bias):
    return pl.pallas_call(
        _router_body,
        out_shape=(
            jax.ShapeDtypeStruct((T, K), jnp.float32),
            jax.ShapeDtypeStruct((T, K), jnp.int32),
            jax.ShapeDtypeStruct((1, NTE), jnp.int32),
        ),
    )(x, Wr, expert_bias.reshape(1, E))


TPW = T // SC_NW


@functools.cache
def _make_scatter_x():
    @functools.partial(
        pl.kernel,
        mesh=plsc.VectorSubcoreMesh(
            core_axis_name="c", subcore_axis_name="s", num_cores=SC_NC,
            num_subcores=SC_NS,
        ),
        out_type=jax.ShapeDtypeStruct((PAD_T, D), jnp.float32),
        scratch_types=[
            pltpu.VMEM((K, TPW), jnp.int32),
            pltpu.VMEM((TPW, D), jnp.float32),
            pltpu.SemaphoreType.DMA,
            pltpu.SemaphoreType.DMA,
            pltpu.SemaphoreType.DMA,
        ],
    )
    def _scatter_x(x_hbm, post_hbm, xs_hbm, idx_v, rows_v, gsem, s0, s1):
        wid = lax.axis_index("s") * SC_NC + lax.axis_index("c")
        pltpu.sync_copy(post_hbm.at[wid], idx_v)
        pltpu.async_copy(x_hbm.at[pl.ds(wid * TPW, TPW)], rows_v, gsem).wait()
        ss = [s0, s1]
        hs = [
            pltpu.async_copy(rows_v, xs_hbm.at[idx_v.at[k]], ss[k & 1])
            for k in range(K)
        ]
        for h in hs:
            h.wait()

    return _scatter_x


def _ffn_body(te_ref, x_ref, wg_hbm, wu_hbm, wd_hbm, y_ref, wgb, wub, wdb, sems):
    i = pl.program_id(0)
    nruns = te_ref[I_NR]
    r = te_ref[I_ROT + i]

    def dmas(rp, slot):
        e = te_ref[I_RUNE + rp]
        return (
            pltpu.make_async_copy(wg_hbm.at[e], wgb.at[slot], sems.at[slot, 0]),
            pltpu.make_async_copy(wu_hbm.at[e], wub.at[slot], sems.at[slot, 1]),
            pltpu.make_async_copy(wd_hbm.at[e], wdb.at[slot], sems.at[slot, 2]),
        )

    def issue(rp):
        for d in dmas(rp, lax.rem(rp, NBUF)):
            d.start()

    def drain(rp):
        for d in dmas(rp, lax.rem(rp, NBUF)):
            d.wait()

    @pl.when(i == 0)
    def _():
        issue(0)

    @pl.when((i == 0) & (nruns > 1))
    def _():
        issue(1)

    @pl.when((i == 0) & (nruns > 2))
    def _():
        issue(2)

    @pl.when(i == 0)
    def _():
        drain(0)

    first = (i > 0) & (r != te_ref[I_ROT + i - 1])

    @pl.when(first & (r + NBUF - 1 < nruns))
    def _():
        issue(r + NBUF - 1)

    @pl.when(first)
    def _():
        drain(r)

    @pl.when(i < te_ref[I_NT])
    def _():
        slot = lax.rem(r, NBUF)
        x = x_ref[...]
        h = jax.nn.silu(_ct(x, wgb[slot])) * _ct(x, wub[slot])
        y_ref[...] = _ct(h, wdb[slot])


def _grouped_ffn(te, xs, Wg, Wu, Wd):
    def clamp(i, te):
        return jnp.minimum(i, te[I_NT] - 1)

    grid_spec = pltpu.PrefetchScalarGridSpec(
        num_scalar_prefetch=1,
        grid=(NT,),
        in_specs=[
            pl.BlockSpec((M, D), lambda i, te: (clamp(i, te), 0)),
            pl.BlockSpec(memory_space=pl.ANY),
            pl.BlockSpec(memory_space=pl.ANY),
            pl.BlockSpec(memory_space=pl.ANY),
        ],
        out_specs=pl.BlockSpec((M, D), lambda i, te: (clamp(i, te), 0)),
        scratch_shapes=[
            pltpu.VMEM((NBUF, FF, D), jnp.float32),
            pltpu.VMEM((NBUF, FF, D), jnp.float32),
            pltpu.VMEM((NBUF, D, FF), jnp.float32),
            pltpu.SemaphoreType.DMA((NBUF, 3)),
        ],
    )
    return pl.pallas_call(
        _ffn_body,
        grid_spec=grid_spec,
        out_shape=jax.ShapeDtypeStruct((PAD_T, D), jnp.float32),
        compiler_params=pltpu.CompilerParams(
            dimension_semantics=("arbitrary",),
        ),
    )(te, xs, Wg, Wu, Wd)


@functools.cache
def _make_gather_y():
    @functools.partial(
        pl.kernel,
        mesh=plsc.VectorSubcoreMesh(
            core_axis_name="c", subcore_axis_name="s", num_cores=SC_NC,
            num_subcores=SC_NS,
        ),
        out_type=jax.ShapeDtypeStruct((K, T, D), jnp.float32),
        scratch_types=[
            pltpu.VMEM((2 * K, TPW // 2), jnp.int32),
            pltpu.VMEM((2, TPW // 2, D), jnp.float32),
            pltpu.SemaphoreType.DMA,
            pltpu.SemaphoreType.DMA,
            pltpu.SemaphoreType.DMA,
            pltpu.SemaphoreType.DMA,
        ],
    )
    def _gather_y(ys_hbm, post_hbm, yp_hbm, idx_v, rows_v, g0, g1, s0, s1):
        wid = lax.axis_index("s") * SC_NC + lax.axis_index("c")
        tok0 = wid * TPW
        half = TPW // 2
        pltpu.sync_copy(post_hbm.at[wid], idx_v)
        gs, ss = [g0, g1], [s0, s1]

        def gather(j, b):
            return pltpu.async_copy(ys_hbm.at[idx_v.at[j]], rows_v.at[b], gs[b])

        def put(j, b):
            k, h = j // 2, j % 2
            return pltpu.async_copy(
                rows_v.at[b], yp_hbm.at[k, pl.ds(tok0 + h * half, half)], ss[b]
            )

        n = 2 * K
        gh, sh = {}, {}
        gh[0] = gather(0, 0)
        for j in range(n):
            b = j & 1
            if j + 1 < n:
                if j - 1 >= 0:
                    sh[j - 1].wait()
                gh[j + 1] = gather(j + 1, 1 - b)
            gh[j].wait()
            sh[j] = put(j, b)
        sh[n - 2].wait()
        sh[n - 1].wait()

    return _gather_y


TB = 256


def _shared_body(x_ref, gs_ref, us_ref, ds_ref, s_ref):
    x = x_ref[...]
    s_ref[...] = _ct(
        jax.nn.silu(_ct(x, gs_ref[...])) * _ct(x, us_ref[...]), ds_ref[...]
    )


def _shared(x, Gs, Us, Ds):
    full = lambda shape: pl.BlockSpec(shape, lambda t: (0,) * len(shape))
    return pl.pallas_call(
        _shared_body,
        grid=(T // TB,),
        in_specs=[
            pl.BlockSpec((TB, D), lambda t: (t, 0)),
            full((FF, D)),
            full((FF, D)),
            full((D, FF)),
        ],
        out_specs=pl.BlockSpec((TB, D), lambda t: (t, 0)),
        out_shape=jax.ShapeDtypeStruct((T, D), jnp.float32),
        compiler_params=pltpu.CompilerParams(
            dimension_semantics=("arbitrary",),
        ),
    )(x, Gs, Us, Ds)


def _combine_body(yp_ref, w_ref, sh_ref, y_ref):
    acc = sh_ref[...]
    for k in range(K):
        acc = acc + w_ref[:, k : k + 1] * yp_ref[k]
    y_ref[...] = acc


def _combine(yp, w, shared):
    return pl.pallas_call(
        _combine_body,
        grid=(T // TB,),
        in_specs=[
            pl.BlockSpec((K, TB, D), lambda t: (0, t, 0)),
            pl.BlockSpec((TB, K), lambda t: (t, 0)),
            pl.BlockSpec((TB, D), lambda t: (t, 0)),
        ],
        out_specs=pl.BlockSpec((TB, D), lambda t: (t, 0)),
        out_shape=jax.ShapeDtypeStruct((T, D), jnp.float32),
        compiler_params=pltpu.CompilerParams(
            dimension_semantics=("arbitrary",),
        ),
    )(yp, w, shared)


_TOK2 = _TOK.reshape(PAIRS // CH, CH)


@jax.jit
def kernel(x, Wr, expert_bias, Wg, Wu, Wd, Gs, Us, Ds):
    w, pos, te = _router_meta(x, Wr, expert_bias)
    post = pos.reshape(SC_NW, TPW, K).transpose(0, 2, 1)
    tef = te.reshape(NTE)
    xs = _make_scatter_x()(x, post)
    ys = _grouped_ffn(tef, xs, Wg, Wu, Wd)
    post16 = post.reshape(SC_NW, 2 * K, TPW // 2)
    yp = _make_gather_y()(ys, post16)
    shared = _shared(x, Gs, Us, Ds)
    return _combine(yp, w, shared)


# speedup vs baseline: 1.0165x; 1.0002x over previous
import functools

import jax
import jax.numpy as jnp
from jax import lax
from jax.experimental import pallas as pl
from jax.experimental.pallas import tpu as pltpu
from jax.experimental.pallas import tpu_sc as plsc

T = 2048
D = 1024
FF = 1024
E = 64
K = 8
ROUTE_SCALE = 2.826

M = 128
PAD_T = 24576
NT = PAD_T // M
NTE = 512
I_NT = 192
I_NR = 193
I_RUNE = 200
I_ROT = 320
NBUF = 3

SC_NC = 2
SC_NS = 16
SC_NW = SC_NC * SC_NS
PAIRS = T * K
PPW = PAIRS // SC_NW


def _ct(a, b):
    return jax.lax.dot_general(
        a, b, (((1,), (1,)), ((), ())), preferred_element_type=jnp.float32
    )


def _mm(a, b):
    return jax.lax.dot_general(
        a, b, (((1,), (0,)), ((), ())), preferred_element_type=jnp.float32
    )


def _router_body(x_ref, wr_ref, bias_ref, w_ref, pos_ref, te_ref):
    x = x_ref[...]
    gates = _ct(x, wr_ref[...])
    scores = jax.nn.sigmoid(gates)
    selection = scores + bias_ref[...]
    iota = lax.broadcasted_iota(jnp.int32, (T, E), 1)
    remaining = selection
    masks = []
    for _ in range(K):
        m = jnp.max(remaining, axis=1, keepdims=True)
        first = jnp.min(jnp.where(remaining == m, iota, E), axis=1, keepdims=True)
        mk = (iota == first).astype(jnp.float32)
        masks.append(mk)
        remaining = jnp.where(mk > 0, -jnp.inf, remaining)
    picked = masks[0]
    for mk in masks[1:]:
        picked = picked + mk
    wsum = jnp.sum(scores * picked, axis=1, keepdims=True)
    wall = scores * (ROUTE_SCALE / wsum)
    w_ref[...] = jnp.concatenate(
        [jnp.sum(mk * wall, axis=1, keepdims=True) for mk in masks], axis=1
    )

    CB = 256
    tri = (
        lax.broadcasted_iota(jnp.int32, (CB, CB), 0)
        > lax.broadcasted_iota(jnp.int32, (CB, CB), 1)
    ).astype(jnp.float32)
    carry = jnp.zeros((1, E), jnp.float32)
    parts = []
    for c in range(T // CB):
        blk = picked[c * CB : (c + 1) * CB, :]
        parts.append(_mm(tri, blk) + carry)
        carry = carry + jnp.sum(blk, axis=0, keepdims=True)
    csum_excl = jnp.concatenate(parts, axis=0)

    pc_row = jnp.ceil(carry / M) * M
    sltE = (
        lax.broadcasted_iota(jnp.int32, (E, E), 0)
        < lax.broadcasted_iota(jnp.int32, (E, E), 1)
    ).astype(jnp.float32)
    po_lane = _mm(pc_row, sltE)
    field = po_lane + csum_excl
    pos_f = jnp.concatenate(
        [jnp.sum(mk * field, axis=1, keepdims=True) for mk in masks], axis=1
    )
    pos_ref[...] = pos_f.astype(jnp.int32)

    ones_t = jnp.ones((T, 1), jnp.float32)
    counts_sub = jax.lax.dot_general(
        picked, ones_t, (((0,), (0,)), ((), ())), preferred_element_type=jnp.float32
    )
    pc_sub = jnp.ceil(counts_sub / M) * M
    slo = (
        lax.broadcasted_iota(jnp.int32, (E, E), 1)
        < lax.broadcasted_iota(jnp.int32, (E, E), 0)
    ).astype(jnp.float32)
    po_sub = _mm(slo, pc_sub)
    lane = lax.broadcasted_iota(jnp.int32, (1, NTE), 1)
    ones_e = jnp.ones((1, E), jnp.float32)
    jm = (lane * M).astype(jnp.float32)
    amat = (po_sub <= jm).astype(jnp.float32)
    te = _mm(ones_e, amat) - 1.0
    nonempty = (counts_sub > 0).astype(jnp.float32)
    sle = (
        lax.broadcasted_iota(jnp.int32, (E, E), 1)
        <= lax.broadcasted_iota(jnp.int32, (E, E), 0)
    ).astype(jnp.float32)
    rank_sub = _mm(sle, nonempty)
    nruns_val = _mm(ones_e, nonempty)
    a_rune = nonempty * (
        (rank_sub - 1.0) == (lane - I_RUNE).astype(jnp.float32)
    ).astype(jnp.float32)
    e_row = lax.broadcasted_iota(jnp.int32, (1, E), 1).astype(jnp.float32)
    rune = _mm(e_row, a_rune)
    jm2 = ((lane - I_ROT) * M).astype(jnp.float32)
    a_rot = nonempty * (po_sub <= jm2).astype(jnp.float32)
    rot = _mm(ones_e, a_rot) - 1.0
    mask63 = (lax.broadcasted_iota(jnp.int32, (E, 1), 0) == E - 1).astype(jnp.float32)
    nt_val = _mm(ones_e, (po_sub + pc_sub) * mask63) / M
    te = jnp.where(lane == I_NT, nt_val, te)
    te = jnp.where(lane == I_NR, nruns_val, te)
    te = jnp.where((lane >= I_RUNE) & (lane < I_RUNE + E), rune, te)
    te = jnp.where(lane >= I_ROT, rot, te)
    te_ref[...] = te.astype(jnp.int32)


def _router_meta(x, Wr, expert_bias):
    return pl.pallas_call(
        _router_body,
        out_shape=(
            jax.ShapeDtypeStruct((T, K), jnp.float32),
            jax.ShapeDtypeStruct((T, K), jnp.int32),
            jax.ShapeDtypeStruct((1, NTE), jnp.int32),
        ),
    )(x, Wr, expert_bias.reshape(1, E))


TPW = T // SC_NW


@functools.cache
def _make_scatter_x():
    @functools.partial(
        pl.kernel,
        mesh=plsc.VectorSubcoreMesh(
            core_axis_name="c", subcore_axis_name="s", num_cores=SC_NC,
            num_subcores=SC_NS,
        ),
        out_type=jax.ShapeDtypeStruct((PAD_T, D), jnp.float32),
        scratch_types=[
            pltpu.VMEM((K, TPW), jnp.int32),
            pltpu.VMEM((TPW, D), jnp.float32),
            pltpu.SemaphoreType.DMA,
            pltpu.SemaphoreType.DMA,
            pltpu.SemaphoreType.DMA,
        ],
    )
    def _scatter_x(x_hbm, post_hbm, xs_hbm, idx_v, rows_v, gsem, s0, s1):
        wid = lax.axis_index("s") * SC_NC + lax.axis_index("c")
        pltpu.sync_copy(post_hbm.at[wid], idx_v)
        pltpu.async_copy(x_hbm.at[pl.ds(wid * TPW, TPW)], rows_v, gsem).wait()
        ss = [s0, s1]
        hs = [
            pltpu.async_copy(rows_v, xs_hbm.at[idx_v.at[k]], ss[k & 1])
            for k in range(K)
        ]
        for h in hs:
            h.wait()

    return _scatter_x


def _ffn_body(te_ref, x_ref, wg_hbm, wu_hbm, wd_hbm, y_ref, wgb, wub, wdb, sems):
    i = pl.program_id(0)
    nruns = te_ref[I_NR]
    r = te_ref[I_ROT + i]

    def dmas(rp, slot):
        e = te_ref[I_RUNE + rp]
        return (
            pltpu.make_async_copy(wg_hbm.at[e], wgb.at[slot], sems.at[slot, 0]),
            pltpu.make_async_copy(wu_hbm.at[e], wub.at[slot], sems.at[slot, 1]),
            pltpu.make_async_copy(wd_hbm.at[e], wdb.at[slot], sems.at[slot, 2]),
        )

    def issue(rp):
        for d in dmas(rp, lax.rem(rp, NBUF)):
            d.start()

    def drain(rp):
        for d in dmas(rp, lax.rem(rp, NBUF)):
            d.wait()

    @pl.when(i == 0)
    def _():
        issue(0)

    @pl.when((i == 0) & (nruns > 1))
    def _():
        issue(1)

    @pl.when((i == 0) & (nruns > 2))
    def _():
        issue(2)

    @pl.when(i == 0)
    def _():
        drain(0)

    first = (i > 0) & (r != te_ref[I_ROT + i - 1])

    @pl.when(first & (r + NBUF - 1 < nruns))
    def _():
        issue(r + NBUF - 1)

    @pl.when(first)
    def _():
        drain(r)

    @pl.when(i < te_ref[I_NT])
    def _():
        slot = lax.rem(r, NBUF)
        x = x_ref[...]
        h = jax.nn.silu(_ct(x, wgb[slot])) * _ct(x, wub[slot])
        y_ref[...] = _ct(h, wdb[slot])


def _grouped_ffn(te, xs, Wg, Wu, Wd):
    def clamp(i, te):
        return jnp.minimum(i, te[I_NT] - 1)

    grid_spec = pltpu.PrefetchScalarGridSpec(
        num_scalar_prefetch=1,
        grid=(NT,),
        in_specs=[
            pl.BlockSpec((M, D), lambda i, te: (clamp(i, te), 0)),
            pl.BlockSpec(memory_space=pl.ANY),
            pl.BlockSpec(memory_space=pl.ANY),
            pl.BlockSpec(memory_space=pl.ANY),
        ],
        out_specs=pl.BlockSpec((M, D), lambda i, te: (clamp(i, te), 0)),
        scratch_shapes=[
            pltpu.VMEM((NBUF, FF, D), jnp.float32),
            pltpu.VMEM((NBUF, FF, D), jnp.float32),
            pltpu.VMEM((NBUF, D, FF), jnp.float32),
            pltpu.SemaphoreType.DMA((NBUF, 3)),
        ],
    )
    return pl.pallas_call(
        _ffn_body,
        grid_spec=grid_spec,
        out_shape=jax.ShapeDtypeStruct((PAD_T, D), jnp.float32),
        compiler_params=pltpu.CompilerParams(
            dimension_semantics=("arbitrary",),
        ),
    )(te, xs, Wg, Wu, Wd)


@functools.cache
def _make_gather_y():
    @functools.partial(
        pl.kernel,
        mesh=plsc.VectorSubcoreMesh(
            core_axis_name="c", subcore_axis_name="s", num_cores=SC_NC,
            num_subcores=SC_NS,
        ),
        out_type=jax.ShapeDtypeStruct((K, T, D), jnp.float32),
        scratch_types=[
            pltpu.VMEM((2 * K, TPW // 2), jnp.int32),
            pltpu.VMEM((2, TPW // 2, D), jnp.float32),
            pltpu.SemaphoreType.DMA,
            pltpu.SemaphoreType.DMA,
            pltpu.SemaphoreType.DMA,
            pltpu.SemaphoreType.DMA,
        ],
    )
    def _gather_y(ys_hbm, post_hbm, yp_hbm, idx_v, rows_v, g0, g1, s0, s1):
        wid = lax.axis_index("s") * SC_NC + lax.axis_index("c")
        tok0 = wid * TPW
        half = TPW // 2
        pltpu.sync_copy(post_hbm.at[wid], idx_v)
        gs, ss = [g0, g1], [s0, s1]

        def gather(j, b):
            return pltpu.async_copy(ys_hbm.at[idx_v.at[j]], rows_v.at[b], gs[b])

        def put(j, b):
            k, h = j // 2, j % 2
            return pltpu.async_copy(
                rows_v.at[b], yp_hbm.at[k, pl.ds(tok0 + h * half, half)], ss[b]
            )

        n = 2 * K
        gh, sh = {}, {}
        gh[0] = gather(0, 0)
        for j in range(n):
            b = j & 1
            if j + 1 < n:
                if j - 1 >= 0:
                    sh[j - 1].wait()
                gh[j + 1] = gather(j + 1, 1 - b)
            gh[j].wait()
            sh[j] = put(j, b)
        sh[n - 2].wait()
        sh[n - 1].wait()

    return _gather_y


TB = 256


def _shared_body(x_ref, gs_ref, us_ref, ds_ref, s_ref):
    x = x_ref[...]
    s_ref[...] = _ct(
        jax.nn.silu(_ct(x, gs_ref[...])) * _ct(x, us_ref[...]), ds_ref[...]
    )


def _shared(x, Gs, Us, Ds):
    full = lambda shape: pl.BlockSpec(shape, lambda t: (0,) * len(shape))
    return pl.pallas_call(
        _shared_body,
        grid=(T // TB,),
        in_specs=[
            pl.BlockSpec((TB, D), lambda t: (t, 0)),
            full((FF, D)),
            full((FF, D)),
            full((D, FF)),
        ],
        out_specs=pl.BlockSpec((TB, D), lambda t: (t, 0)),
        out_shape=jax.ShapeDtypeStruct((T, D), jnp.float32),
        compiler_params=pltpu.CompilerParams(
            dimension_semantics=("arbitrary",),
        ),
    )(x, Gs, Us, Ds)


def _combine_body(yp_ref, w_ref, sh_ref, y_ref):
    acc = sh_ref[...]
    for k in range(K):
        acc = acc + w_ref[:, k : k + 1] * yp_ref[k]
    y_ref[...] = acc


def _combine(yp, w, shared):
    return pl.pallas_call(
        _combine_body,
        grid=(T // TB,),
        in_specs=[
            pl.BlockSpec((K, TB, D), lambda t: (0, t, 0)),
            pl.BlockSpec((TB, K), lambda t: (t, 0)),
            pl.BlockSpec((TB, D), lambda t: (t, 0)),
        ],
        out_specs=pl.BlockSpec((TB, D), lambda t: (t, 0)),
        out_shape=jax.ShapeDtypeStruct((T, D), jnp.float32),
        compiler_params=pltpu.CompilerParams(
            dimension_semantics=("arbitrary",),
        ),
    )(yp, w, shared)


@jax.jit
def kernel(x, Wr, expert_bias, Wg, Wu, Wd, Gs, Us, Ds):
    w, pos, te = _router_meta(x, Wr, expert_bias)
    post = pos.reshape(SC_NW, TPW, K).transpose(0, 2, 1)
    tef = te.reshape(NTE)
    xs = _make_scatter_x()(x, post)
    ys = _grouped_ffn(tef, xs, Wg, Wu, Wd)
    post16 = post.reshape(SC_NW, 2 * K, TPW // 2)
    yp = _make_gather_y()(ys, post16)
    shared = _shared(x, Gs, Us, Ds)
    return _combine(yp, w, shared)
